# Initial kernel scaffold; baseline (speedup 1.0000x reference)
import jax
import jax.numpy as jnp
from jax import lax
from jax.experimental import pallas as pl
from jax.experimental.pallas import tpu as pltpu
from jax.experimental.pallas import tpu_sc as plsc

DIM = 64
BATCH = 4096
HIST = 50
TOTAL = BATCH * HIST
NC, NS = 2, 16
NW = NC * NS
ROWS_PER_TILE = TOTAL // NW
CHUNK = 128
NCHUNK = ROWS_PER_TILE // CHUNK


def _gather_body(idx_hbm, table_hbm, out_hbm, idx_v, buf, gsem):
    wid = lax.axis_index("s") * NC + lax.axis_index("c")
    pltpu.sync_copy(idx_hbm.at[pl.ds(wid * NCHUNK, NCHUNK)], idx_v)
    base = wid * ROWS_PER_TILE

    def body(j, carry):
        pltpu.async_copy(table_hbm.at[idx_v.at[j]], buf, gsem).wait()
        pltpu.sync_copy(buf, out_hbm.at[pl.ds(base + j * CHUNK, CHUNK)])
        return carry

    lax.fori_loop(0, NCHUNK, body, 0, unroll=False)


def kernel(inputs, kernel, pretrained):
    table = jnp.concatenate((kernel, pretrained), axis=0)
    idx = inputs.reshape(-1).astype(jnp.int32).reshape(NW * NCHUNK, CHUNK)
    mesh = plsc.VectorSubcoreMesh(core_axis_name="c", subcore_axis_name="s")
    out = pl.kernel(
        _gather_body,
        mesh=mesh,
        out_type=jax.ShapeDtypeStruct((TOTAL, DIM), jnp.float32),
        scratch_types=[
            pltpu.VMEM((NCHUNK, CHUNK), jnp.int32),
            pltpu.VMEM((CHUNK, DIM), jnp.float32),
            pltpu.SemaphoreType.DMA,
        ],
    )(idx, table)
    return out.reshape(BATCH, HIST, DIM)


# speedup vs baseline: 3.7922x; 3.7922x over previous
import jax
import jax.numpy as jnp
from jax import lax
from jax.experimental import pallas as pl
from jax.experimental.pallas import tpu as pltpu
from jax.experimental.pallas import tpu_sc as plsc

DIM = 64
BATCH = 4096
HIST = 50
TOTAL = BATCH * HIST
NC, NS = 2, 16
NW = NC * NS
ROWS_PER_TILE = TOTAL // NW
CHUNK = 128
NCHUNK = ROWS_PER_TILE // CHUNK


def _gather_body(idx_hbm, table_hbm, out_hbm, idx_v, buf, gsem):
    wid = lax.axis_index("s") * NC + lax.axis_index("c")
    pltpu.sync_copy(idx_hbm.at[wid], idx_v)
    base = wid * ROWS_PER_TILE

    def body(j, carry):
        pltpu.async_copy(table_hbm.at[idx_v.at[j]], buf, gsem).wait()
        pltpu.sync_copy(buf, out_hbm.at[pl.ds(base + j * CHUNK, CHUNK)])
        return carry

    lax.fori_loop(0, NCHUNK, body, 0, unroll=False)


def kernel(inputs, kernel, pretrained):
    table = jnp.concatenate((kernel, pretrained), axis=0)
    idx = inputs.reshape(-1).astype(jnp.int32).reshape(NW, NCHUNK, CHUNK)
    mesh = plsc.VectorSubcoreMesh(core_axis_name="c", subcore_axis_name="s")
    out = pl.kernel(
        _gather_body,
        mesh=mesh,
        compiler_params=pltpu.CompilerParams(use_tc_tiling_on_sc=False),
        out_type=jax.ShapeDtypeStruct((TOTAL, DIM), jnp.float32),
        scratch_types=[
            pltpu.VMEM((NCHUNK, CHUNK), jnp.int32),
            pltpu.VMEM((CHUNK, DIM), jnp.float32),
            pltpu.SemaphoreType.DMA,
        ],
    )(idx, table)
    return out.reshape(BATCH, HIST, DIM)


# speedup vs baseline: 4.2714x; 1.1264x over previous
import jax
import jax.numpy as jnp
from jax import lax
from jax.experimental import pallas as pl
from jax.experimental.pallas import tpu as pltpu
from jax.experimental.pallas import tpu_sc as plsc

DIM = 64
BATCH = 4096
HIST = 50
TOTAL = BATCH * HIST
NC, NS = 2, 16
NW = NC * NS
ROWS_PER_TILE = TOTAL // NW
CHUNK = 128
NCHUNK = ROWS_PER_TILE // CHUNK


NBUF = 5
AHEAD = NBUF - 1
NOUTER = NCHUNK // NBUF


def _gather_body(idx_hbm, table_hbm, out_hbm, idx_v, bufs, gsem):
    wid = lax.axis_index("s") * NC + lax.axis_index("c")
    pltpu.sync_copy(idx_hbm.at[wid], idx_v)
    base = wid * ROWS_PER_TILE

    def fire(j, b):
        pltpu.async_copy(table_hbm.at[idx_v.at[j]], bufs.at[b], gsem.at[b])

    for jp in range(AHEAD):
        fire(jp, jp)

    def body(j0, carry):
        for b in range(NBUF):
            j = j0 * NBUF + b
            pltpu.make_async_copy(
                table_hbm.at[idx_v.at[j]], bufs.at[b], gsem.at[b]
            ).wait()
            pltpu.sync_copy(bufs.at[b], out_hbm.at[pl.ds(base + j * CHUNK, CHUNK)])
            k = j + AHEAD
            bk = (b + AHEAD) % NBUF

            @pl.when(k < NCHUNK)
            def _():
                fire(k, bk)
        return carry

    lax.fori_loop(0, NOUTER, body, 0, unroll=False)


def kernel(inputs, kernel, pretrained):
    table = jnp.concatenate((kernel, pretrained), axis=0)
    idx = inputs.reshape(-1).astype(jnp.int32).reshape(NW, NCHUNK, CHUNK)
    mesh = plsc.VectorSubcoreMesh(core_axis_name="c", subcore_axis_name="s")
    out = pl.kernel(
        _gather_body,
        mesh=mesh,
        compiler_params=pltpu.CompilerParams(use_tc_tiling_on_sc=False),
        out_type=jax.ShapeDtypeStruct((TOTAL, DIM), jnp.float32),
        scratch_types=[
            pltpu.VMEM((NCHUNK, CHUNK), jnp.int32),
            pltpu.VMEM((NBUF, CHUNK, DIM), jnp.float32),
            pltpu.SemaphoreType.DMA((NBUF,)),
        ],
    )(idx, table)
    return out.reshape(BATCH, HIST, DIM)


# speedup vs baseline: 4.7614x; 1.1147x over previous
import jax
import jax.numpy as jnp
from jax import lax
from jax.experimental import pallas as pl
from jax.experimental.pallas import tpu as pltpu
from jax.experimental.pallas import tpu_sc as plsc

DIM = 64
BATCH = 4096
HIST = 50
TOTAL = BATCH * HIST
NC, NS = 2, 16
NW = NC * NS
ROWS_PER_TILE = TOTAL // NW
CHUNK = 128
NCHUNK = ROWS_PER_TILE // CHUNK
NBUF = 5
AHEAD = NBUF - 1
NOUTER = NCHUNK // NBUF
NG = CHUNK // 16


def _gather_body(idx_hbm, krows_hbm, table_hbm, out_hbm,
                 idx_v, sidx_v, kv, bufs, minv, gsem):
    wid = lax.axis_index("s") * NC + lax.axis_index("c")
    base = wid * ROWS_PER_TILE
    pltpu.sync_copy(idx_hbm.at[pl.ds(base, ROWS_PER_TILE)], idx_v)
    pltpu.sync_copy(krows_hbm, kv)

    def prep(j, carry):
        sl0 = pl.ds(j * CHUNK, 16)
        acc = idx_v[sl0]
        sidx_v[sl0] = jnp.maximum(acc - 2, 0)
        for g in range(1, NG):
            sl = pl.ds(j * CHUNK + g * 16, 16)
            v = idx_v[sl]
            acc = jnp.minimum(acc, v)
            sidx_v[sl] = jnp.maximum(v - 2, 0)
        minv[pl.ds(j * 16, 16)] = acc
        return carry

    lax.fori_loop(0, NCHUNK, prep, 0, unroll=False)

    def fire(j, b):
        pltpu.async_copy(
            table_hbm.at[sidx_v.at[pl.ds(j * CHUNK, CHUNK)]],
            bufs.at[b], gsem.at[b])

    for jp in range(AHEAD):
        fire(jp, jp)

    def body(j0, carry):
        for b in range(NBUF):
            j = j0 * NBUF + b
            pltpu.make_async_copy(
                table_hbm.at[sidx_v.at[pl.ds(j * CHUNK, CHUNK)]],
                bufs.at[b], gsem.at[b],
            ).wait()

            pltpu.sync_copy(bufs.at[b], out_hbm.at[pl.ds(base + j * CHUNK, CHUNK)])
            k = j + AHEAD
            bk = (b + AHEAD) % NBUF

            @pl.when(k < NCHUNK)
            def _(k=k, bk=bk):
                fire(k, bk)
        return carry

    lax.fori_loop(0, NOUTER, body, 0, unroll=False)

    def patch(j, carry):
        mv = minv[pl.ds(j * 16, 16)]
        mmin = mv[0]
        for l in range(1, 16):
            mmin = jnp.minimum(mmin, mv[l])

        @pl.when(mmin < 2)
        def _(j=j):
            def g_body(g, carry2):
                v = idx_v[pl.ds(j * CHUNK + g * 16, 16)]
                for l in range(16):
                    s = v[l]

                    @pl.when(s < 2)
                    def __(s=s, l=l):
                        pltpu.sync_copy(
                            kv.at[pl.ds(s, 1)],
                            out_hbm.at[pl.ds(base + j * CHUNK + g * 16 + l, 1)])
                return carry2

            lax.fori_loop(0, NG, g_body, 0, unroll=False)
        return carry

    lax.fori_loop(0, NCHUNK, patch, 0, unroll=False)


def kernel(inputs, kernel, pretrained):
    idx = inputs.reshape(-1).astype(jnp.int32)
    mesh = plsc.VectorSubcoreMesh(core_axis_name="c", subcore_axis_name="s")
    out = pl.kernel(
        _gather_body,
        mesh=mesh,
        compiler_params=pltpu.CompilerParams(use_tc_tiling_on_sc=False),
        out_type=jax.ShapeDtypeStruct((TOTAL, DIM), jnp.float32),
        scratch_types=[
            pltpu.VMEM((ROWS_PER_TILE,), jnp.int32),
            pltpu.VMEM((ROWS_PER_TILE,), jnp.int32),
            pltpu.VMEM((2, DIM), jnp.float32),
            pltpu.VMEM((NBUF, CHUNK, DIM), jnp.float32),
            pltpu.VMEM((NCHUNK * 16,), jnp.int32),
            pltpu.SemaphoreType.DMA((NBUF,)),
        ],
    )(idx, kernel, pretrained)
    return out.reshape(BATCH, HIST, DIM)
